# baseline (speedup 1.0000x reference)
import functools

import jax
import jax.numpy as jnp
from jax import lax
from jax.experimental import pallas as pl
from jax.experimental.pallas import tpu as pltpu
from jax.experimental.pallas import tpu_sc as plsc

_N = 16384
_C = 128
_NC = 2
_NS = 16
_NW = _NC * _NS
_ROWS_PER_W = _N // _NW
_CHUNK = 128
_LANES = 16

_sc_mesh = plsc.VectorSubcoreMesh(core_axis_name="c", subcore_axis_name="s",
                                  num_cores=_NC, num_subcores=_NS)


@functools.partial(
    pl.kernel,
    out_type=jax.ShapeDtypeStruct((6 * _N, _C), jnp.float32),
    mesh=_sc_mesh,
    scratch_types=[
        pltpu.VMEM((_CHUNK,), jnp.int32),
        pltpu.VMEM((_CHUNK, _C), jnp.float32),
        pltpu.VMEM((2, _C), jnp.float32),
        pltpu.SemaphoreType.DMA,
    ],
    compiler_params=pltpu.CompilerParams(use_tc_tiling_on_sc=True),
)
def _sc_cat(users_idx, items_idx, users_emb, items_emb, bias2,
            out, idx_v, rows_v, bias_v, sem):
    wid = lax.axis_index("s") * _NC + lax.axis_index("c")
    pltpu.sync_copy(bias2, bias_v)
    base = wid * _ROWS_PER_W
    for t in range(2):
        idx_hbm = users_idx if t == 0 else items_idx
        emb = users_emb if t == 0 else items_emb
        out_base = (1 + 3 * t) * _N
        bias_regs = [bias_v[t, pl.ds(cc * _LANES, _LANES)] for cc in range(_C // _LANES)]
        for ch in range(_ROWS_PER_W // _CHUNK):
            r0 = base + ch * _CHUNK
            pltpu.sync_copy(idx_hbm.at[pl.ds(r0, _CHUNK)], idx_v)
            pltpu.async_copy(emb.at[idx_v], rows_v, sem).wait()

            def _add_bias(r, carry):
                for cc in range(_C // _LANES):
                    sl = pl.ds(cc * _LANES, _LANES)
                    rows_v[r, sl] = rows_v[r, sl] + bias_regs[cc]
                return carry

            lax.fori_loop(0, _CHUNK, _add_bias, 0, unroll=2)
            pltpu.sync_copy(rows_v, out.at[pl.ds(out_base + r0, _CHUNK)])


_BR = 1024
_NI = _N // _BR
_HALF = _NI // 2


def _tc_body(buf, unum, inum, utext, itext, utW, itW, nw2, nb2, bias4, out_ref):
    j = pl.program_id(0)
    del buf

    def num_block(t, num_ref):
        x = num_ref[0, :]
        z = x[:, None] * nw2[t, :][None, :] + nb2[t, :][None, :]
        sig = 1.0 / (1.0 + jnp.exp(-z))
        out_ref[...] = z * sig + bias4[2 * t, :][None, :]

    def text_block(t, text_ref, w_ref):
        acc = jnp.dot(text_ref[:, 0, :].astype(jnp.bfloat16),
                      w_ref[...].astype(jnp.bfloat16),
                      preferred_element_type=jnp.float32)
        out_ref[...] = acc + bias4[2 * t + 1, :][None, :]

    @pl.when(j == 0)
    def _():
        num_block(0, unum)

    @pl.when(j == 1)
    def _():
        text_block(0, utext, utW)

    @pl.when(j == 2)
    def _():
        num_block(1, inum)

    @pl.when(j == 3)
    def _():
        text_block(1, itext, itW)


def _out_map(j, i):
    return (j * (3 * _HALF) + (j % 2) * _HALF + i, 0)


_tc_call = pl.pallas_call(
    _tc_body,
    grid=(4, _NI),
    in_specs=[
        pl.BlockSpec(memory_space=pl.ANY),
        pl.BlockSpec((1, _BR), lambda j, i: (0, jnp.where(j == 0, i, 0))),
        pl.BlockSpec((1, _BR), lambda j, i: (0, jnp.where(j == 2, i, 0))),
        pl.BlockSpec((_BR, 1, 300), lambda j, i: (jnp.where(j == 1, i, 0), 0, 0)),
        pl.BlockSpec((_BR, 1, 300), lambda j, i: (jnp.where(j == 3, i, 0), 0, 0)),
        pl.BlockSpec((300, _C), lambda j, i: (0, 0)),
        pl.BlockSpec((300, _C), lambda j, i: (0, 0)),
        pl.BlockSpec((2, _C), lambda j, i: (0, 0)),
        pl.BlockSpec((2, _C), lambda j, i: (0, 0)),
        pl.BlockSpec((4, _C), lambda j, i: (0, 0)),
    ],
    out_specs=pl.BlockSpec((_BR, _C), _out_map),
    out_shape=jax.ShapeDtypeStruct((6 * _N, _C), jnp.float32),
    input_output_aliases={0: 0},
)


def kernel(users_num, users_cat, users_text, items_num, items_cat, items_text, table_emb,
           users_num_W, users_num_b, users_num_col, users_cat_emb, users_cat_col,
           users_text_W, users_text_b, users_text_col,
           items_num_W, items_num_b, items_num_col, items_cat_emb, items_cat_col,
           items_text_W, items_text_b, items_text_col):
    temb0 = table_emb[0]
    temb1 = table_emb[1]
    cat_bias = jnp.stack([users_cat_col + temb0, items_cat_col + temb1])
    buf = _sc_cat(users_cat.reshape(_N), items_cat.reshape(_N),
                  users_cat_emb, items_cat_emb, cat_bias)

    nw2 = jnp.concatenate([users_num_W, items_num_W], axis=0)
    nb2 = jnp.stack([users_num_b, items_num_b])
    bias4 = jnp.stack([
        users_num_col + temb0,
        users_text_b + users_text_col + temb0,
        items_num_col + temb1,
        items_text_b + items_text_col + temb1,
    ])
    x = _tc_call(buf,
                 users_num.reshape(1, _N), items_num.reshape(1, _N),
                 users_text, items_text,
                 users_text_W, items_text_W, nw2, nb2, bias4)

    ar = jnp.arange(_N, dtype=jnp.int32)
    node = jnp.concatenate([ar, ar, ar, ar + _N, ar + _N, ar + _N])
    col = jnp.repeat(jnp.arange(6, dtype=jnp.int32), _N)
    table = jnp.repeat(jnp.arange(2, dtype=jnp.int32), 3 * _N)
    return x, node, col, table


# speedup vs baseline: 1.0427x; 1.0427x over previous
import functools

import jax
import jax.numpy as jnp
from jax import lax
from jax.experimental import pallas as pl
from jax.experimental.pallas import tpu as pltpu
from jax.experimental.pallas import tpu_sc as plsc

_N = 16384
_C = 128
_NC = 2
_NS = 16
_NW = _NC * _NS
_ROWS_PER_W = _N // _NW
_CHUNK = 128
_LANES = 16

_sc_mesh = plsc.VectorSubcoreMesh(core_axis_name="c", subcore_axis_name="s",
                                  num_cores=_NC, num_subcores=_NS)


@functools.partial(
    pl.kernel,
    out_type=jax.ShapeDtypeStruct((6 * _N, _C), jnp.float32),
    mesh=_sc_mesh,
    scratch_types=[
        pltpu.VMEM((_CHUNK,), jnp.int32),
        pltpu.VMEM((_CHUNK, _C), jnp.float32),
        pltpu.VMEM((2, _C), jnp.float32),
        pltpu.SemaphoreType.DMA,
    ],
    compiler_params=pltpu.CompilerParams(use_tc_tiling_on_sc=True),
)
def _sc_cat(users_idx, items_idx, users_emb, items_emb, bias2,
            out, idx_v, rows_v, bias_v, sem):
    wid = lax.axis_index("s") * _NC + lax.axis_index("c")
    pltpu.sync_copy(bias2, bias_v)
    base = wid * _ROWS_PER_W
    for t in range(2):
        idx_hbm = users_idx if t == 0 else items_idx
        emb = users_emb if t == 0 else items_emb
        out_base = (1 + 3 * t) * _N
        bias_regs = [bias_v[t, pl.ds(cc * _LANES, _LANES)] for cc in range(_C // _LANES)]
        for ch in range(_ROWS_PER_W // _CHUNK):
            r0 = base + ch * _CHUNK
            pltpu.sync_copy(idx_hbm.at[pl.ds(r0, _CHUNK)], idx_v)
            pltpu.async_copy(emb.at[idx_v], rows_v, sem).wait()

            def _add_bias(r, carry):
                for cc in range(_C // _LANES):
                    sl = pl.ds(cc * _LANES, _LANES)
                    rows_v[r, sl] = rows_v[r, sl] + bias_regs[cc]
                return carry

            lax.fori_loop(0, _CHUNK, _add_bias, 0, unroll=2)
            pltpu.sync_copy(rows_v, out.at[pl.ds(out_base + r0, _CHUNK)])


_BR = 512
_NI = _N // _BR
_BRN = 2048
_NIN = _N // _BRN


def _text_body(buf, text, w, bias, out_ref):
    del buf
    acc = jnp.dot(text[:, 0, :], w[...], preferred_element_type=jnp.float32)
    out_ref[...] = acc + bias[0, :][None, :]


def _make_text_call(base_block):
    return pl.pallas_call(
        _text_body,
        grid=(_NI,),
        in_specs=[
            pl.BlockSpec(memory_space=pl.ANY),
            pl.BlockSpec((_BR, 1, 300), lambda i: (i, 0, 0)),
            pl.BlockSpec((300, _C), lambda i: (0, 0)),
            pl.BlockSpec((1, _C), lambda i: (0, 0)),
        ],
        out_specs=pl.BlockSpec((_BR, _C), lambda i: (base_block + i, 0)),
        out_shape=jax.ShapeDtypeStruct((6 * _N, _C), jnp.float32),
        input_output_aliases={0: 0},
    )


_text_users = _make_text_call(2 * _NI)
_text_items = _make_text_call(5 * _NI)


def _num_body(buf, num, w, b_in, b_out, out_ref):
    del buf
    x = num[0, 0, :]
    z = x[:, None] * w[0, 0, :][None, :] + b_in[0, 0, :][None, :]
    sig = 1.0 / (1.0 + jnp.exp(-z))
    out_ref[...] = z * sig + b_out[0, 0, :][None, :]


_num_call = pl.pallas_call(
    _num_body,
    grid=(2, _NIN),
    in_specs=[
        pl.BlockSpec(memory_space=pl.ANY),
        pl.BlockSpec((1, 1, _BRN), lambda t, i: (t, 0, i)),
        pl.BlockSpec((1, 1, _C), lambda t, i: (t, 0, 0)),
        pl.BlockSpec((1, 1, _C), lambda t, i: (t, 0, 0)),
        pl.BlockSpec((1, 1, _C), lambda t, i: (t, 0, 0)),
    ],
    out_specs=pl.BlockSpec((_BRN, _C), lambda t, i: (t * 3 * _NIN + i, 0)),
    out_shape=jax.ShapeDtypeStruct((6 * _N, _C), jnp.float32),
    input_output_aliases={0: 0},
)


def kernel(users_num, users_cat, users_text, items_num, items_cat, items_text, table_emb,
           users_num_W, users_num_b, users_num_col, users_cat_emb, users_cat_col,
           users_text_W, users_text_b, users_text_col,
           items_num_W, items_num_b, items_num_col, items_cat_emb, items_cat_col,
           items_text_W, items_text_b, items_text_col):
    temb0 = table_emb[0]
    temb1 = table_emb[1]
    cat_bias = jnp.stack([users_cat_col + temb0, items_cat_col + temb1])
    buf = _sc_cat(users_cat.reshape(_N), items_cat.reshape(_N),
                  users_cat_emb, items_cat_emb, cat_bias)

    x = _text_users(buf, users_text, users_text_W,
                    (users_text_b + users_text_col + temb0).reshape(1, _C))
    x = _text_items(x, items_text, items_text_W,
                    (items_text_b + items_text_col + temb1).reshape(1, _C))
    num2 = jnp.stack([users_num[:, 0], items_num[:, 0]]).reshape(2, 1, _N)
    nw2 = jnp.stack([users_num_W, items_num_W])
    nb2 = jnp.stack([users_num_b, items_num_b]).reshape(2, 1, _C)
    nbias2 = jnp.stack([users_num_col + temb0,
                        items_num_col + temb1]).reshape(2, 1, _C)
    x = _num_call(x, num2, nw2, nb2, nbias2)

    ar = jnp.arange(_N, dtype=jnp.int32)
    node = jnp.concatenate([ar, ar, ar, ar + _N, ar + _N, ar + _N])
    col = jnp.repeat(jnp.arange(6, dtype=jnp.int32), _N)
    table = jnp.repeat(jnp.arange(2, dtype=jnp.int32), 3 * _N)
    return x, node, col, table


# speedup vs baseline: 2.2393x; 2.1476x over previous
import functools

import jax
import jax.numpy as jnp
from jax import lax
from jax.experimental import pallas as pl
from jax.experimental.pallas import tpu as pltpu
from jax.experimental.pallas import tpu_sc as plsc

_N = 16384
_C = 128
_NC = 2
_NS = 16
_NW = _NC * _NS
_ROWS_PER_W = _N // _NW
_CHUNK = 128
_LANES = 16

_sc_mesh = plsc.VectorSubcoreMesh(core_axis_name="c", subcore_axis_name="s",
                                  num_cores=_NC, num_subcores=_NS)


@functools.partial(
    pl.kernel,
    out_type=jax.ShapeDtypeStruct((6 * _N, _C), jnp.float32),
    mesh=_sc_mesh,
    scratch_types=[
        pltpu.VMEM((_CHUNK,), jnp.int32),
        pltpu.VMEM((_CHUNK, _C), jnp.float32),
        pltpu.VMEM((2, _C), jnp.float32),
        pltpu.SemaphoreType.DMA,
    ],
    compiler_params=pltpu.CompilerParams(use_tc_tiling_on_sc=True),
)
def _sc_cat(users_idx, items_idx, users_emb, items_emb, bias2,
            out, idx_v, rows_v, bias_v, sem):
    wid = lax.axis_index("s") * _NC + lax.axis_index("c")
    pltpu.sync_copy(bias2, bias_v)
    base = wid * _ROWS_PER_W
    for t in range(2):
        idx_hbm = users_idx if t == 0 else items_idx
        emb = users_emb if t == 0 else items_emb
        out_base = (1 + 3 * t) * _N
        bias_regs = [bias_v[t, pl.ds(cc * _LANES, _LANES)] for cc in range(_C // _LANES)]
        for ch in range(_ROWS_PER_W // _CHUNK):
            r0 = base + ch * _CHUNK
            pltpu.sync_copy(idx_hbm.at[pl.ds(r0, _CHUNK)], idx_v)
            pltpu.async_copy(emb.at[idx_v], rows_v, sem).wait()

            def _add_bias(r, carry):
                for cc in range(_C // _LANES):
                    sl = pl.ds(cc * _LANES, _LANES)
                    rows_v[r, sl] = rows_v[r, sl] + bias_regs[cc]
                return carry

            lax.fori_loop(0, _CHUNK, _add_bias, 0, unroll=2)
            pltpu.sync_copy(rows_v, out.at[pl.ds(out_base + r0, _CHUNK)])


_BR = 512
_NI = _N // _BR
_BRN = 2048
_NIN = _N // _BRN


def _text_body(buf, text, w, bias, out_ref):
    del buf
    lhs_t = text[:, 0, :]
    acc = jax.lax.dot_general(lhs_t, w[...],
                              dimension_numbers=(((0,), (0,)), ((), ())),
                              preferred_element_type=jnp.float32)
    out_ref[...] = acc + bias[0, :][None, :]


def _make_text_call(base_block):
    return pl.pallas_call(
        _text_body,
        grid=(_NI,),
        in_specs=[
            pl.BlockSpec(memory_space=pl.ANY),
            pl.BlockSpec((300, 1, _BR), lambda i: (0, 0, i)),
            pl.BlockSpec((300, _C), lambda i: (0, 0)),
            pl.BlockSpec((1, _C), lambda i: (0, 0)),
        ],
        out_specs=pl.BlockSpec((_BR, _C), lambda i: (base_block + i, 0)),
        out_shape=jax.ShapeDtypeStruct((6 * _N, _C), jnp.float32),
        input_output_aliases={0: 0},
    )


_text_users = _make_text_call(2 * _NI)
_text_items = _make_text_call(5 * _NI)


def _num_body(buf, num, w, b_in, b_out, out_ref):
    del buf
    x = num[0, 0, :]
    z = x[:, None] * w[0, 0, :][None, :] + b_in[0, 0, :][None, :]
    sig = 1.0 / (1.0 + jnp.exp(-z))
    out_ref[...] = z * sig + b_out[0, 0, :][None, :]


_num_call = pl.pallas_call(
    _num_body,
    grid=(2, _NIN),
    in_specs=[
        pl.BlockSpec(memory_space=pl.ANY),
        pl.BlockSpec((1, 1, _BRN), lambda t, i: (t, 0, i)),
        pl.BlockSpec((1, 1, _C), lambda t, i: (t, 0, 0)),
        pl.BlockSpec((1, 1, _C), lambda t, i: (t, 0, 0)),
        pl.BlockSpec((1, 1, _C), lambda t, i: (t, 0, 0)),
    ],
    out_specs=pl.BlockSpec((_BRN, _C), lambda t, i: (t * 3 * _NIN + i, 0)),
    out_shape=jax.ShapeDtypeStruct((6 * _N, _C), jnp.float32),
    input_output_aliases={0: 0},
)


def kernel(users_num, users_cat, users_text, items_num, items_cat, items_text, table_emb,
           users_num_W, users_num_b, users_num_col, users_cat_emb, users_cat_col,
           users_text_W, users_text_b, users_text_col,
           items_num_W, items_num_b, items_num_col, items_cat_emb, items_cat_col,
           items_text_W, items_text_b, items_text_col):
    temb0 = table_emb[0]
    temb1 = table_emb[1]
    cat_bias = jnp.stack([users_cat_col + temb0, items_cat_col + temb1])
    buf = _sc_cat(users_cat.reshape(_N), items_cat.reshape(_N),
                  users_cat_emb, items_cat_emb, cat_bias)

    x = _text_users(buf, jnp.transpose(users_text, (2, 1, 0)), users_text_W,
                    (users_text_b + users_text_col + temb0).reshape(1, _C))
    x = _text_items(x, jnp.transpose(items_text, (2, 1, 0)), items_text_W,
                    (items_text_b + items_text_col + temb1).reshape(1, _C))
    num2 = jnp.stack([users_num[:, 0], items_num[:, 0]]).reshape(2, 1, _N)
    nw2 = jnp.stack([users_num_W, items_num_W])
    nb2 = jnp.stack([users_num_b, items_num_b]).reshape(2, 1, _C)
    nbias2 = jnp.stack([users_num_col + temb0,
                        items_num_col + temb1]).reshape(2, 1, _C)
    x = _num_call(x, num2, nw2, nb2, nbias2)

    ar = jnp.arange(_N, dtype=jnp.int32)
    node = jnp.concatenate([ar, ar, ar, ar + _N, ar + _N, ar + _N])
    col = jnp.repeat(jnp.arange(6, dtype=jnp.int32), _N)
    table = jnp.repeat(jnp.arange(2, dtype=jnp.int32), 3 * _N)
    return x, node, col, table
